# baseline (speedup 1.0000x reference)
import functools

import jax
import jax.numpy as jnp
from jax import lax
from jax.experimental import pallas as pl
from jax.experimental.pallas import tpu as pltpu
from jax.experimental.pallas import tpu_sc as plsc

_VOCAB = 1000000
_EMB = 128
_B = 4096
_NDENSE = 13
_NSPARSE = 26
_BOT = 256

_NC, _NS = 2, 16
_NW = _NC * _NS
_TOT = _B * _NSPARSE
_PERW = _TOT // _NW
_CHUNK = 128
_NCHUNK = _PERW // _CHUNK


def _sc_gather(table, xs_flat):
  mesh = plsc.VectorSubcoreMesh(
      core_axis_name="c", subcore_axis_name="s",
      num_cores=_NC, num_subcores=_NS)

  @functools.partial(
      pl.kernel,
      out_type=jax.ShapeDtypeStruct((_TOT, _EMB), jnp.float32),
      mesh=mesh,
      scratch_types=[
          pltpu.VMEM((_PERW,), jnp.float32),
          pltpu.VMEM((_NCHUNK, _CHUNK), jnp.int32),
          pltpu.VMEM((2, _CHUNK, _EMB), jnp.float32),
          pltpu.SemaphoreType.DMA,
          pltpu.SemaphoreType.DMA,
          pltpu.SemaphoreType.DMA,
          pltpu.SemaphoreType.DMA,
      ],
  )
  def gather_kernel(table_hbm, xs_hbm, out_hbm, xv, idxv, bufs,
                    gs0, gs1, cs0, cs1):
    wid = lax.axis_index("s") * _NC + lax.axis_index("c")
    base = wid * _PERW
    pltpu.sync_copy(xs_hbm.at[pl.ds(base, _PERW)], xv)
    gsems = (gs0, gs1)
    csems = (cs0, cs1)

    def conv(j, carry):
      for i in range(_CHUNK // 16):
        v = xv[pl.ds(j * _CHUNK + i * 16, 16)]
        idxv[j, pl.ds(i * 16, 16)] = v.astype(jnp.int32) % _VOCAB
      return carry

    lax.fori_loop(0, _NCHUNK, conv, 0)

    def chunk_step(j, b, wait_prev):
      if wait_prev:
        pltpu.make_async_copy(
            bufs.at[b],
            out_hbm.at[pl.ds(base + (j - 2) * _CHUNK, _CHUNK)],
            csems[b]).wait()
      pltpu.async_copy(table_hbm.at[idxv.at[j]], bufs.at[b], gsems[b]).wait()
      pltpu.async_copy(
          bufs.at[b], out_hbm.at[pl.ds(base + j * _CHUNK, _CHUNK)], csems[b])

    for b in range(2):
      chunk_step(b, b, False)

    def pair(i, carry):
      for b in range(2):
        chunk_step(2 * i + b, b, True)
      return carry

    lax.fori_loop(1, _NCHUNK // 2, pair, 0)

    for b in range(2):
      pltpu.make_async_copy(
          bufs.at[b],
          out_hbm.at[pl.ds(base + (_NCHUNK - 2 + b) * _CHUNK, _CHUNK)],
          csems[b]).wait()

  return gather_kernel(table, xs_flat)


def _mlp_body(dense_ref, emb_ref,
              bw0, bb0, bw1, bb1, bw2, bb2,
              tw0, tb0, tw1, tb1, tw2, tb2, tw3, tb3, tw4, tb4,
              out_ref):
  f32 = jnp.float32
  bf16 = jnp.bfloat16

  def bdot(a, w):
    return jnp.dot(a.astype(bf16), w.astype(bf16), preferred_element_type=f32)

  def fdot(a, w):
    return jnp.dot(a, w, preferred_element_type=f32)

  dense = dense_ref[...]
  h = jax.nn.relu(fdot(dense, bw0[...]) + bb0[...])
  h = jax.nn.relu(fdot(h, bw1[...]) + bb1[...]) + h
  h = jax.nn.relu(fdot(h, bw2[...]) + bb2[...]) + h
  t = (fdot(h, tw0[:_BOT, :])
       + bdot(emb_ref[...], tw0[_BOT:, :])
       + tb0[...])
  t = jax.nn.relu(t)
  t = jax.nn.relu(fdot(t, tw1[...]) + tb1[...]) + t
  t = jax.nn.relu(fdot(t, tw2[...]) + tb2[...]) + t
  t = jax.nn.relu(fdot(t, tw3[...]) + tb3[...]) + t
  out_ref[...] = fdot(t, tw4[...]) + tb4[...]


def _tc_mlp(dense, emb, bw0, bb0, bw1, bb1, bw2, bb2,
            tw0, tb0, tw1, tb1, tw2, tb2, tw3, tb3, tw4, tb4):
  bb = 512
  grid = (_B // bb,)

  def full(w):
    return pl.BlockSpec(w.shape, lambda i: (0,) * w.ndim)

  weights = (bw0, bb0, bw1, bb1, bw2, bb2,
             tw0, tb0, tw1, tb1, tw2, tb2, tw3, tb3, tw4, tb4)
  return pl.pallas_call(
      _mlp_body,
      grid=grid,
      in_specs=[
          pl.BlockSpec((bb, _NDENSE), lambda i: (i, 0)),
          pl.BlockSpec((bb, _NSPARSE * _EMB), lambda i: (i, 0)),
      ] + [full(w) for w in weights],
      out_specs=pl.BlockSpec((bb, 1), lambda i: (i, 0)),
      out_shape=jax.ShapeDtypeStruct((_B, 1), jnp.float32),
  )(dense, emb, *weights)


def kernel(x, emb_table, bot_w0, bot_b0, bot_w1, bot_b1, bot_w2, bot_b2,
           top_w0, top_b0, top_w1, top_b1, top_w2, top_b2,
           top_w3, top_b3, top_w4, top_b4):
  dense = x[:, :_NDENSE]
  xs_flat = x[:, _NDENSE:].reshape(-1)
  emb_rows = _sc_gather(emb_table, xs_flat)
  emb = emb_rows.reshape(_B, _NSPARSE * _EMB)
  return _tc_mlp(
      dense, emb,
      bot_w0, bot_b0.reshape(1, -1), bot_w1, bot_b1.reshape(1, -1),
      bot_w2, bot_b2.reshape(1, -1),
      top_w0, top_b0.reshape(1, -1), top_w1, top_b1.reshape(1, -1),
      top_w2, top_b2.reshape(1, -1), top_w3, top_b3.reshape(1, -1),
      top_w4, top_b4.reshape(1, -1))


# speedup vs baseline: 1.4458x; 1.4458x over previous
import functools

import jax
import jax.numpy as jnp
from jax import lax
from jax.experimental import pallas as pl
from jax.experimental.pallas import tpu as pltpu
from jax.experimental.pallas import tpu_sc as plsc

_VOCAB = 1000000
_EMB = 128
_B = 4096
_NDENSE = 13
_NSPARSE = 26
_BOT = 256

_NC, _NS = 2, 16
_NW = _NC * _NS
_TOT = _B * _NSPARSE
_PERW = _TOT // _NW
_CHUNK = 128
_NCHUNK = _PERW // _CHUNK
_BCHUNKS = _B // _CHUNK


def _sc_gather(table, xs_flat):
  mesh = plsc.VectorSubcoreMesh(
      core_axis_name="c", subcore_axis_name="s",
      num_cores=_NC, num_subcores=_NS)

  @functools.partial(
      pl.kernel,
      out_type=jax.ShapeDtypeStruct((_NSPARSE, _B, _EMB), jnp.float32),
      mesh=mesh,
      scratch_types=[
          pltpu.VMEM((_PERW,), jnp.float32),
          pltpu.VMEM((_NCHUNK, _CHUNK), jnp.int32),
          pltpu.VMEM((2, _CHUNK, _EMB), jnp.float32),
          pltpu.SemaphoreType.DMA,
          pltpu.SemaphoreType.DMA,
          pltpu.SemaphoreType.DMA,
          pltpu.SemaphoreType.DMA,
      ],
  )
  def gather_kernel(table_hbm, xs_hbm, out_hbm, xv, idxv, bufs,
                    gs0, gs1, cs0, cs1):
    wid = lax.axis_index("s") * _NC + lax.axis_index("c")
    base = wid * _PERW
    pltpu.sync_copy(xs_hbm.at[pl.ds(base, _PERW)], xv)
    gsems = (gs0, gs1)
    csems = (cs0, cs1)

    def conv(j, carry):
      for i in range(_CHUNK // 16):
        v = xv[pl.ds(j * _CHUNK + i * 16, 16)]
        idxv[j, pl.ds(i * 16, 16)] = v.astype(jnp.int32) % _VOCAB
      return carry

    lax.fori_loop(0, _NCHUNK, conv, 0)

    def out_slice(j):
      c = wid * _NCHUNK + j
      f = c // _BCHUNKS
      b0 = (c % _BCHUNKS) * _CHUNK
      return out_hbm.at[f, pl.ds(b0, _CHUNK)]

    def chunk_step(j, b, wait_prev):
      if wait_prev:
        pltpu.make_async_copy(bufs.at[b], out_slice(j - 2), csems[b]).wait()
      pltpu.async_copy(table_hbm.at[idxv.at[j]], bufs.at[b], gsems[b]).wait()
      pltpu.async_copy(bufs.at[b], out_slice(j), csems[b])

    for b in range(2):
      chunk_step(b, b, False)

    def pair(i, carry):
      for b in range(2):
        chunk_step(2 * i + b, b, True)
      return carry

    lax.fori_loop(1, _NCHUNK // 2, pair, 0)

    for b in range(2):
      pltpu.make_async_copy(
          bufs.at[b], out_slice(_NCHUNK - 2 + b), csems[b]).wait()

  return gather_kernel(table, xs_flat)


def _mlp_body(dense_ref, emb_ref,
              bw0, bb0, bw1, bb1, bw2, bb2,
              twh, twe, tb0, tw1, tb1, tw2, tb2, tw3, tb3, tw4, tb4,
              out_ref):
  f32 = jnp.float32
  bf16 = jnp.bfloat16

  def bdot(a, w):
    return jnp.dot(a.astype(bf16), w.astype(bf16), preferred_element_type=f32)

  def fdot(a, w):
    return jnp.dot(a, w, preferred_element_type=f32)

  dense = dense_ref[...]
  h = jax.nn.relu(fdot(dense, bw0[...]) + bb0[...])
  h = jax.nn.relu(fdot(h, bw1[...]) + bb1[...]) + h
  h = jax.nn.relu(fdot(h, bw2[...]) + bb2[...]) + h
  t = fdot(h, twh[...]) + tb0[...]
  for f in range(_NSPARSE):
    t += bdot(emb_ref[f], twe[f])
  t = jax.nn.relu(t)
  t = jax.nn.relu(fdot(t, tw1[...]) + tb1[...]) + t
  t = jax.nn.relu(fdot(t, tw2[...]) + tb2[...]) + t
  t = jax.nn.relu(fdot(t, tw3[...]) + tb3[...]) + t
  out_ref[...] = fdot(t, tw4[...]) + tb4[...]


def _tc_mlp(dense, emb3, bw0, bb0, bw1, bb1, bw2, bb2,
            twh, twe, tb0, tw1, tb1, tw2, tb2, tw3, tb3, tw4, tb4):
  bb = 512
  grid = (_B // bb,)

  def full(w):
    return pl.BlockSpec(w.shape, lambda i: (0,) * w.ndim)

  weights = (bw0, bb0, bw1, bb1, bw2, bb2,
             twh, twe, tb0, tw1, tb1, tw2, tb2, tw3, tb3, tw4, tb4)
  return pl.pallas_call(
      _mlp_body,
      grid=grid,
      in_specs=[
          pl.BlockSpec((bb, _NDENSE), lambda i: (i, 0)),
          pl.BlockSpec((_NSPARSE, bb, _EMB), lambda i: (0, i, 0)),
      ] + [full(w) for w in weights],
      out_specs=pl.BlockSpec((bb, 1), lambda i: (i, 0)),
      out_shape=jax.ShapeDtypeStruct((_B, 1), jnp.float32),
  )(dense, emb3, *weights)


def kernel(x, emb_table, bot_w0, bot_b0, bot_w1, bot_b1, bot_w2, bot_b2,
           top_w0, top_b0, top_w1, top_b1, top_w2, top_b2,
           top_w3, top_b3, top_w4, top_b4):
  dense = x[:, :_NDENSE]
  xs_flat = x[:, _NDENSE:].T.reshape(-1)
  emb3 = _sc_gather(emb_table, xs_flat)
  return _tc_mlp(
      dense, emb3,
      bot_w0, bot_b0.reshape(1, -1), bot_w1, bot_b1.reshape(1, -1),
      bot_w2, bot_b2.reshape(1, -1),
      top_w0[:_BOT], top_w0[_BOT:].reshape(_NSPARSE, _EMB, _BOT),
      top_b0.reshape(1, -1), top_w1, top_b1.reshape(1, -1),
      top_w2, top_b2.reshape(1, -1), top_w3, top_b3.reshape(1, -1),
      top_w4, top_b4.reshape(1, -1))


# speedup vs baseline: 1.4774x; 1.0219x over previous
import functools

import jax
import jax.numpy as jnp
from jax import lax
from jax.experimental import pallas as pl
from jax.experimental.pallas import tpu as pltpu
from jax.experimental.pallas import tpu_sc as plsc

_VOCAB = 1000000
_EMB = 128
_B = 4096
_NDENSE = 13
_NSPARSE = 26
_BOT = 256

_NC, _NS = 2, 16
_NW = _NC * _NS
_TOT = _B * _NSPARSE
_PERW = _TOT // _NW
_CHUNK = 128
_NCHUNK = _PERW // _CHUNK
_BCHUNKS = _B // _CHUNK


def _sc_gather(table, xs_flat, nb):
  perw = nb * _NSPARSE // _NW
  nchunk = perw // _CHUNK
  bchunks = nb // _CHUNK
  mesh = plsc.VectorSubcoreMesh(
      core_axis_name="c", subcore_axis_name="s",
      num_cores=_NC, num_subcores=_NS)

  @functools.partial(
      pl.kernel,
      out_type=jax.ShapeDtypeStruct((_NSPARSE, nb, _EMB), jnp.float32),
      mesh=mesh,
      scratch_types=[
          pltpu.VMEM((perw,), jnp.float32),
          pltpu.VMEM((nchunk, _CHUNK), jnp.int32),
          pltpu.VMEM((2, _CHUNK, _EMB), jnp.float32),
          pltpu.SemaphoreType.DMA,
          pltpu.SemaphoreType.DMA,
          pltpu.SemaphoreType.DMA,
          pltpu.SemaphoreType.DMA,
      ],
  )
  def gather_kernel(table_hbm, xs_hbm, out_hbm, xv, idxv, bufs,
                    gs0, gs1, cs0, cs1):
    wid = lax.axis_index("s") * _NC + lax.axis_index("c")
    base = wid * perw
    pltpu.sync_copy(xs_hbm.at[pl.ds(base, perw)], xv)
    gsems = (gs0, gs1)
    csems = (cs0, cs1)

    def conv(j, carry):
      for i in range(_CHUNK // 16):
        v = xv[pl.ds(j * _CHUNK + i * 16, 16)]
        idxv[j, pl.ds(i * 16, 16)] = v.astype(jnp.int32) % _VOCAB
      return carry

    lax.fori_loop(0, nchunk, conv, 0)

    def out_slice(j):
      c = wid * nchunk + j
      f = c // bchunks
      b0 = (c % bchunks) * _CHUNK
      return out_hbm.at[f, pl.ds(b0, _CHUNK)]

    def chunk_step(j, b, wait_prev):
      if wait_prev:
        pltpu.make_async_copy(bufs.at[b], out_slice(j - 2), csems[b]).wait()
      pltpu.async_copy(table_hbm.at[idxv.at[j]], bufs.at[b], gsems[b]).wait()
      pltpu.async_copy(bufs.at[b], out_slice(j), csems[b])

    for b in range(2):
      chunk_step(b, b, False)

    def pair(i, carry):
      for b in range(2):
        chunk_step(2 * i + b, b, True)
      return carry

    lax.fori_loop(1, nchunk // 2, pair, 0)

    for j in range(2 * (nchunk // 2), nchunk):
      chunk_step(j, j % 2, True)

    for k in (nchunk - 2, nchunk - 1):
      pltpu.make_async_copy(bufs.at[k % 2], out_slice(k), csems[k % 2]).wait()

  return gather_kernel(table, xs_flat)


def _mlp_body(dense_ref, emb_ref,
              bw0, bb0, bw1, bb1, bw2, bb2,
              twh, twe, tb0, tw1, tb1, tw2, tb2, tw3, tb3, tw4, tb4,
              out_ref):
  f32 = jnp.float32
  bf16 = jnp.bfloat16

  def bdot(a, w):
    return jnp.dot(a.astype(bf16), w.astype(bf16), preferred_element_type=f32)

  def fdot(a, w):
    return jnp.dot(a, w, preferred_element_type=f32)

  dense = dense_ref[...]
  h = jax.nn.relu(fdot(dense, bw0[...]) + bb0[...])
  h = jax.nn.relu(fdot(h, bw1[...]) + bb1[...]) + h
  h = jax.nn.relu(fdot(h, bw2[...]) + bb2[...]) + h
  t = fdot(h, twh[...]) + tb0[...]
  for f in range(_NSPARSE):
    t += bdot(emb_ref[f], twe[f])
  t = jax.nn.relu(t)
  t = jax.nn.relu(fdot(t, tw1[...]) + tb1[...]) + t
  t = jax.nn.relu(fdot(t, tw2[...]) + tb2[...]) + t
  t = jax.nn.relu(fdot(t, tw3[...]) + tb3[...]) + t
  out_ref[...] = fdot(t, tw4[...]) + tb4[...]


def _tc_mlp(dense, emb3, bw0, bb0, bw1, bb1, bw2, bb2,
            twh, twe, tb0, tw1, tb1, tw2, tb2, tw3, tb3, tw4, tb4):
  nb = dense.shape[0]
  bb = 512
  grid = (nb // bb,)

  def full(w):
    return pl.BlockSpec(w.shape, lambda i: (0,) * w.ndim)

  weights = (bw0, bb0, bw1, bb1, bw2, bb2,
             twh, twe, tb0, tw1, tb1, tw2, tb2, tw3, tb3, tw4, tb4)
  return pl.pallas_call(
      _mlp_body,
      grid=grid,
      in_specs=[
          pl.BlockSpec((bb, _NDENSE), lambda i: (i, 0)),
          pl.BlockSpec((_NSPARSE, bb, _EMB), lambda i: (0, i, 0)),
      ] + [full(w) for w in weights],
      out_specs=pl.BlockSpec((bb, 1), lambda i: (i, 0)),
      out_shape=jax.ShapeDtypeStruct((nb, 1), jnp.float32),
  )(dense, emb3, *weights)


def kernel(x, emb_table, bot_w0, bot_b0, bot_w1, bot_b1, bot_w2, bot_b2,
           top_w0, top_b0, top_w1, top_b1, top_w2, top_b2,
           top_w3, top_b3, top_w4, top_b4):
  dense = x[:, :_NDENSE]
  xst = x[:, _NDENSE:].T
  nsplit = 2
  nb = _B // nsplit
  weights = (
      bot_w0, bot_b0.reshape(1, -1), bot_w1, bot_b1.reshape(1, -1),
      bot_w2, bot_b2.reshape(1, -1),
      top_w0[:_BOT], top_w0[_BOT:].reshape(_NSPARSE, _EMB, _BOT),
      top_b0.reshape(1, -1), top_w1, top_b1.reshape(1, -1),
      top_w2, top_b2.reshape(1, -1), top_w3, top_b3.reshape(1, -1),
      top_w4, top_b4.reshape(1, -1))
  embs = [_sc_gather(emb_table, xst[:, s * nb:(s + 1) * nb].reshape(-1), nb)
          for s in range(nsplit)]
  outs = [_tc_mlp(dense[s * nb:(s + 1) * nb], embs[s], *weights)
          for s in range(nsplit)]
  return jnp.concatenate(outs, axis=0)


# speedup vs baseline: 1.5139x; 1.0247x over previous
import functools

import jax
import jax.numpy as jnp
import numpy as np
from jax import lax
from jax.experimental import pallas as pl
from jax.experimental.pallas import tpu as pltpu
from jax.experimental.pallas import tpu_sc as plsc

_VOCAB = 1000000
_EMB = 128
_B = 4096
_NDENSE = 13
_NSPARSE = 26
_BOT = 256

_NC, _NS = 2, 16
_NW = _NC * _NS
_TOT = _B * _NSPARSE
_PERW = _TOT // _NW
_CHUNK = 128
_NCHUNK = _PERW // _CHUNK
_BCHUNKS = _B // _CHUNK

_J = np.arange(2 * _EMB)
_S = _J // _EMB
_WC = _J % _EMB
_FOFF = _WC // (_EMB // 2)
_JJ = _WC % (_EMB // 2)
_ELEM = (_JJ // 16) * 32 + _S * 16 + (_JJ % 16)


def _twe2(tw0e):
  rows = 2 * np.arange(_NSPARSE // 2)[:, None] + _FOFF[None, :]
  elems = np.broadcast_to(_ELEM, rows.shape)
  return tw0e[rows, elems]


def _sc_gather(table, xs_flat, nb):
  perw = nb * _NSPARSE // _NW
  nchunk = perw // _CHUNK
  bchunks = nb // _CHUNK
  mesh = plsc.VectorSubcoreMesh(
      core_axis_name="c", subcore_axis_name="s",
      num_cores=_NC, num_subcores=_NS)

  @functools.partial(
      pl.kernel,
      out_type=jax.ShapeDtypeStruct((_NSPARSE // 2, nb, _EMB), jnp.int32),
      mesh=mesh,
      scratch_types=[
          pltpu.VMEM((nchunk, _CHUNK), jnp.float32),
          pltpu.VMEM((nchunk, _CHUNK), jnp.int32),
          pltpu.VMEM((2, _CHUNK, _EMB), jnp.float32),
          pltpu.VMEM((2, _CHUNK // 2, _EMB), jnp.int32),
          pltpu.SemaphoreType.DMA,
          pltpu.SemaphoreType.DMA,
          pltpu.SemaphoreType.DMA,
          pltpu.SemaphoreType.DMA,
          pltpu.SemaphoreType.DMA,
      ],
  )
  def gather_kernel(table_hbm, xs_hbm, out_hbm, xv, idxv, bufs, cbufs,
                    gs0, gs1, cs0, cs1, xsem):
    wid = lax.axis_index("s") * _NC + lax.axis_index("c")
    gsems = (gs0, gs1)
    csems = (cs0, cs1)
    btiles = nb // (_CHUNK // 2)

    def tile_coords(j):
      t = wid * nchunk + j
      fp = t // btiles
      b0 = (t % btiles) * (_CHUNK // 2)
      return fp, b0

    id_copies = []
    for j in range(nchunk):
      fp, b0 = tile_coords(j)
      p0 = 2 * fp * nb + b0
      id_copies.append(pltpu.async_copy(
          xs_hbm.at[pl.ds(p0, _CHUNK // 2)],
          xv.at[j, pl.ds(0, _CHUNK // 2)], xsem))
      id_copies.append(pltpu.async_copy(
          xs_hbm.at[pl.ds(p0 + nb, _CHUNK // 2)],
          xv.at[j, pl.ds(_CHUNK // 2, _CHUNK // 2)], xsem))
    for c in id_copies:
      c.wait()

    def conv(j, carry):
      for i in range(_CHUNK // 16):
        v = xv[j, pl.ds(i * 16, 16)]
        idxv[j, pl.ds(i * 16, 16)] = v.astype(jnp.int32) % _VOCAB
      return carry

    lax.fori_loop(0, nchunk, conv, 0)

    def out_slice(j):
      fp, b0 = tile_coords(j)
      return out_hbm.at[fp, pl.ds(b0, _CHUNK // 2)]

    def start_gather(j, b):
      pltpu.async_copy(table_hbm.at[idxv.at[j]], bufs.at[b], gsems[b])

    def pack_rows(b):
      def row(r, carry):
        for src_half in range(2):
          src = r + src_half * (_CHUNK // 2)
          for g in range(_EMB // 32):
            lo = bufs[b, src, pl.ds(g * 32, 16)]
            hi = bufs[b, src, pl.ds(g * 32 + 16, 16)]
            ul = lax.bitcast_convert_type(lo, jnp.int32)
            uh = lax.bitcast_convert_type(hi, jnp.int32)
            wl = lax.shift_right_logical(ul + jnp.int32(0x8000), 16)
            wh = (uh + jnp.int32(0x8000)) & jnp.int32(-0x10000)
            cbufs[b, r, pl.ds(src_half * (_EMB // 2) + g * 16, 16)] = wl | wh
        return carry

      lax.fori_loop(0, _CHUNK // 2, row, 0)

    def chunk_step(j, b, prefetch_next, wait_prev):
      pltpu.make_async_copy(
          table_hbm.at[idxv.at[j]], bufs.at[b], gsems[b]).wait()
      if prefetch_next:
        start_gather(j + 1, 1 - b)
      if wait_prev:
        pltpu.make_async_copy(cbufs.at[b], out_slice(j - 2), csems[b]).wait()
      pack_rows(b)
      pltpu.async_copy(cbufs.at[b], out_slice(j), csems[b])

    start_gather(0, 0)
    chunk_step(0, 0, True, False)
    chunk_step(1, 1, nchunk > 2, False)

    def pair(i, carry):
      for b in range(2):
        chunk_step(2 * i + b, b, True, True)
      return carry

    lax.fori_loop(1, (nchunk - 1) // 2, pair, 0)

    for j in range(2 * ((nchunk - 1) // 2), nchunk):
      if j >= 2:
        chunk_step(j, j % 2, j + 1 < nchunk, True)

    for k in (nchunk - 2, nchunk - 1):
      pltpu.make_async_copy(cbufs.at[k % 2], out_slice(k), csems[k % 2]).wait()

  return gather_kernel(table, xs_flat)


def _mlp_body(dense_ref, emb_ref,
              bw0, bb0, bw1, bb1, bw2, bb2,
              twh, twe, tb0, tw1, tb1, tw2, tb2, tw3, tb3, tw4, tb4,
              out_ref):
  f32 = jnp.float32
  bf16 = jnp.bfloat16

  def bdot(a, w):
    return jnp.dot(a.astype(bf16), w.astype(bf16), preferred_element_type=f32)

  def fdot(a, w):
    return jnp.dot(a, w, preferred_element_type=f32)

  dense = dense_ref[...]
  h = jax.nn.relu(fdot(dense, bw0[...]) + bb0[...])
  h = jax.nn.relu(fdot(h, bw1[...]) + bb1[...]) + h
  h = jax.nn.relu(fdot(h, bw2[...]) + bb2[...]) + h
  t = fdot(h, twh[...]) + tb0[...]
  for fp in range(_NSPARSE // 2):
    w = emb_ref[fp]
    lo = lax.bitcast_convert_type(w << 16, jnp.float32)
    hi = lax.bitcast_convert_type(w & jnp.int32(-0x10000), jnp.float32)
    t += bdot(jnp.concatenate([lo, hi], axis=1), twe[fp])
  t = jax.nn.relu(t)
  t = jax.nn.relu(fdot(t, tw1[...]) + tb1[...]) + t
  t = jax.nn.relu(fdot(t, tw2[...]) + tb2[...]) + t
  t = jax.nn.relu(fdot(t, tw3[...]) + tb3[...]) + t
  out_ref[...] = fdot(t, tw4[...]) + tb4[...]


def _tc_mlp(dense, emb3, bw0, bb0, bw1, bb1, bw2, bb2,
            twh, twe, tb0, tw1, tb1, tw2, tb2, tw3, tb3, tw4, tb4):
  nb = dense.shape[0]
  bb = 512
  grid = (nb // bb,)

  def full(w):
    return pl.BlockSpec(w.shape, lambda i: (0,) * w.ndim)

  weights = (bw0, bb0, bw1, bb1, bw2, bb2,
             twh, twe, tb0, tw1, tb1, tw2, tb2, tw3, tb3, tw4, tb4)
  return pl.pallas_call(
      _mlp_body,
      grid=grid,
      in_specs=[
          pl.BlockSpec((bb, _NDENSE), lambda i: (i, 0)),
          pl.BlockSpec((_NSPARSE // 2, bb, _EMB), lambda i: (0, i, 0)),
      ] + [full(w) for w in weights],
      out_specs=pl.BlockSpec((bb, 1), lambda i: (i, 0)),
      out_shape=jax.ShapeDtypeStruct((nb, 1), jnp.float32),
  )(dense, emb3, *weights)


def kernel(x, emb_table, bot_w0, bot_b0, bot_w1, bot_b1, bot_w2, bot_b2,
           top_w0, top_b0, top_w1, top_b1, top_w2, top_b2,
           top_w3, top_b3, top_w4, top_b4):
  dense = x[:, :_NDENSE]
  xst = x[:, _NDENSE:].T
  nsplit = 2
  nb = _B // nsplit
  weights = (
      bot_w0, bot_b0.reshape(1, -1), bot_w1, bot_b1.reshape(1, -1),
      bot_w2, bot_b2.reshape(1, -1),
      top_w0[:_BOT], _twe2(top_w0[_BOT:].reshape(_NSPARSE, _EMB, _BOT)),
      top_b0.reshape(1, -1), top_w1, top_b1.reshape(1, -1),
      top_w2, top_b2.reshape(1, -1), top_w3, top_b3.reshape(1, -1),
      top_w4, top_b4.reshape(1, -1))
  embs = [_sc_gather(emb_table, xst[:, s * nb:(s + 1) * nb].reshape(-1), nb)
          for s in range(nsplit)]
  outs = [_tc_mlp(dense[s * nb:(s + 1) * nb], embs[s], *weights)
          for s in range(nsplit)]
  return jnp.concatenate(outs, axis=0)


# speedup vs baseline: 1.6443x; 1.0861x over previous
import functools

import jax
import jax.numpy as jnp
import numpy as np
from jax import lax
from jax.experimental import pallas as pl
from jax.experimental.pallas import tpu as pltpu
from jax.experimental.pallas import tpu_sc as plsc

_VOCAB = 1000000
_EMB = 128
_B = 4096
_NDENSE = 13
_NSPARSE = 26
_BOT = 256

_NC, _NS = 2, 16
_NW = _NC * _NS
_TOT = _B * _NSPARSE
_PERW = _TOT // _NW
_CHUNK = 128
_NCHUNK = _PERW // _CHUNK
_BCHUNKS = _B // _CHUNK

def _twe2(tw0e):
  t = tw0e.reshape(_NSPARSE // 2, 2, 4, 2, 16, _BOT)
  return t.transpose(0, 3, 1, 2, 4, 5).reshape(_NSPARSE // 2, 2 * _EMB, _BOT)


def _sc_gather(table, xst, boff, nb):
  nchunk = nb * _NSPARSE // _NW // _CHUNK
  mesh = plsc.VectorSubcoreMesh(
      core_axis_name="c", subcore_axis_name="s",
      num_cores=_NC, num_subcores=_NS)

  @functools.partial(
      pl.kernel,
      out_type=jax.ShapeDtypeStruct((_NSPARSE // 2, nb, _EMB), jnp.int32),
      mesh=mesh,
      scratch_types=[
          pltpu.VMEM((nchunk, _CHUNK), jnp.float32),
          pltpu.VMEM((nchunk, _CHUNK), jnp.int32),
          pltpu.VMEM((2, _CHUNK, _EMB), jnp.float32),
          pltpu.VMEM((2, _CHUNK // 2, _EMB), jnp.int32),
          pltpu.SemaphoreType.DMA,
          pltpu.SemaphoreType.DMA,
          pltpu.SemaphoreType.DMA,
          pltpu.SemaphoreType.DMA,
          pltpu.SemaphoreType.DMA,
      ],
  )
  def gather_kernel(table_hbm, xs_hbm, out_hbm, xv, idxv, bufs, cbufs,
                    gs0, gs1, cs0, cs1, xsem):
    wid = lax.axis_index("s") * _NC + lax.axis_index("c")
    gsems = (gs0, gs1)
    csems = (cs0, cs1)
    btiles = nb // (_CHUNK // 2)

    def tile_coords(j):
      t = wid * nchunk + j
      fp = t // btiles
      b0 = (t % btiles) * (_CHUNK // 2)
      return fp, b0

    id_copies = []
    for j in range(nchunk):
      fp, b0 = tile_coords(j)
      for s in range(2):
        id_copies.append(pltpu.async_copy(
            xs_hbm.at[2 * fp + s, pl.ds(boff + b0, _CHUNK // 2)],
            xv.at[j, pl.ds(s * (_CHUNK // 2), _CHUNK // 2)], xsem))
    for c in id_copies:
      c.wait()

    def conv(j, carry):
      for i in range(_CHUNK // 16):
        v = xv[j, pl.ds(i * 16, 16)]
        idxv[j, pl.ds(i * 16, 16)] = v.astype(jnp.int32) % _VOCAB
      return carry

    lax.fori_loop(0, nchunk, conv, 0)

    def out_slice(j):
      fp, b0 = tile_coords(j)
      return out_hbm.at[fp, pl.ds(b0, _CHUNK // 2)]

    def start_gather(j, b):
      pltpu.async_copy(table_hbm.at[idxv.at[j]], bufs.at[b], gsems[b])

    def pack_rows(b):
      def row(r, carry):
        for src_half in range(2):
          src = r + src_half * (_CHUNK // 2)
          for g in range(_EMB // 32):
            lo = bufs[b, src, pl.ds(g * 32, 16)]
            hi = bufs[b, src, pl.ds(g * 32 + 16, 16)]
            ul = lax.bitcast_convert_type(lo, jnp.int32)
            uh = lax.bitcast_convert_type(hi, jnp.int32)
            wl = lax.shift_right_logical(ul + jnp.int32(0x8000), 16)
            wh = (uh + jnp.int32(0x8000)) & jnp.int32(-0x10000)
            cbufs[b, r, pl.ds(src_half * (_EMB // 2) + g * 16, 16)] = wl | wh
        return carry

      lax.fori_loop(0, _CHUNK // 2, row, 0)

    def chunk_step(j, b, prefetch_next, wait_prev):
      pltpu.make_async_copy(
          table_hbm.at[idxv.at[j]], bufs.at[b], gsems[b]).wait()
      if prefetch_next:
        start_gather(j + 1, 1 - b)
      if wait_prev:
        pltpu.make_async_copy(cbufs.at[b], out_slice(j - 2), csems[b]).wait()
      pack_rows(b)
      pltpu.async_copy(cbufs.at[b], out_slice(j), csems[b])

    start_gather(0, 0)
    chunk_step(0, 0, True, False)
    chunk_step(1, 1, nchunk > 2, False)

    def pair(i, carry):
      for b in range(2):
        chunk_step(2 * i + b, b, True, True)
      return carry

    lax.fori_loop(1, (nchunk - 1) // 2, pair, 0)

    for j in range(2 * ((nchunk - 1) // 2), nchunk):
      if j >= 2:
        chunk_step(j, j % 2, j + 1 < nchunk, True)

    for k in (nchunk - 2, nchunk - 1):
      pltpu.make_async_copy(cbufs.at[k % 2], out_slice(k), csems[k % 2]).wait()

  return gather_kernel(table, xst)


def _mlp_body(dense_ref, emb_ref,
              bw0, bb0, bw1, bb1, bw2, bb2,
              twh, twe, tb0, tw1, tb1, tw2, tb2, tw3, tb3, tw4, tb4,
              out_ref):
  f32 = jnp.float32
  bf16 = jnp.bfloat16

  def bdot(a, w):
    return jnp.dot(a.astype(bf16), w.astype(bf16), preferred_element_type=f32)

  def fdot(a, w):
    return jnp.dot(a, w, preferred_element_type=f32)

  dense = dense_ref[...]
  h = jax.nn.relu(fdot(dense, bw0[...]) + bb0[...])
  h = jax.nn.relu(fdot(h, bw1[...]) + bb1[...]) + h
  h = jax.nn.relu(fdot(h, bw2[...]) + bb2[...]) + h
  t = fdot(h, twh[...]) + tb0[...]
  for fp in range(_NSPARSE // 2):
    w = emb_ref[fp]
    lo = lax.bitcast_convert_type(w << 16, jnp.float32)
    hi = lax.bitcast_convert_type(w & jnp.int32(-0x10000), jnp.float32)
    t += bdot(jnp.concatenate([lo, hi], axis=1), twe[fp])
  t = jax.nn.relu(t)
  t = jax.nn.relu(fdot(t, tw1[...]) + tb1[...]) + t
  t = jax.nn.relu(fdot(t, tw2[...]) + tb2[...]) + t
  t = jax.nn.relu(fdot(t, tw3[...]) + tb3[...]) + t
  out_ref[...] = fdot(t, tw4[...]) + tb4[...]


def _tc_mlp(dense, emb3, bw0, bb0, bw1, bb1, bw2, bb2,
            twh, twe, tb0, tw1, tb1, tw2, tb2, tw3, tb3, tw4, tb4):
  nb = dense.shape[0]
  bb = 512
  grid = (nb // bb,)

  def full(w):
    return pl.BlockSpec(w.shape, lambda i: (0,) * w.ndim)

  weights = (bw0, bb0, bw1, bb1, bw2, bb2,
             twh, twe, tb0, tw1, tb1, tw2, tb2, tw3, tb3, tw4, tb4)
  return pl.pallas_call(
      _mlp_body,
      grid=grid,
      in_specs=[
          pl.BlockSpec((bb, _NDENSE), lambda i: (i, 0)),
          pl.BlockSpec((_NSPARSE // 2, bb, _EMB), lambda i: (0, i, 0)),
      ] + [full(w) for w in weights],
      out_specs=pl.BlockSpec((bb, 1), lambda i: (i, 0)),
      out_shape=jax.ShapeDtypeStruct((nb, 1), jnp.float32),
  )(dense, emb3, *weights)


def kernel(x, emb_table, bot_w0, bot_b0, bot_w1, bot_b1, bot_w2, bot_b2,
           top_w0, top_b0, top_w1, top_b1, top_w2, top_b2,
           top_w3, top_b3, top_w4, top_b4):
  dense = x[:, :_NDENSE]
  xst = x[:, _NDENSE:].T
  nsplit = 2
  nb = _B // nsplit
  weights = (
      bot_w0, bot_b0.reshape(1, -1), bot_w1, bot_b1.reshape(1, -1),
      bot_w2, bot_b2.reshape(1, -1),
      top_w0[:_BOT], _twe2(top_w0[_BOT:]),
      top_b0.reshape(1, -1), top_w1, top_b1.reshape(1, -1),
      top_w2, top_b2.reshape(1, -1), top_w3, top_b3.reshape(1, -1),
      top_w4, top_b4.reshape(1, -1))
  embs = [_sc_gather(emb_table, xst, s * nb, nb) for s in range(nsplit)]
  outs = [_tc_mlp(dense[s * nb:(s + 1) * nb], embs[s], *weights)
          for s in range(nsplit)]
  return jnp.concatenate(outs, axis=0)
